# Initial kernel scaffold; baseline (speedup 1.0000x reference)
import jax
import jax.numpy as jnp
from jax.experimental import pallas as pl


def kernel(u, v, batch):
    raise NotImplementedError("write your pallas kernel here")


# speedup vs baseline: 4.5746x; 4.5746x over previous
import jax
import jax.numpy as jnp
from jax import lax
from jax.experimental import pallas as pl
from jax.experimental.pallas import tpu as pltpu
from jax.experimental.pallas import tpu_sc as plsc

NC = 2
NS = 16
NW = NC * NS

S = 10000
SP = 10240
N = 320000
D = 128

C = 80
RPW = N // NW
NCHUNK = RPW // C
RPS = SP // NS
PCH = 160
NPCH = RPS // PCH


def _sc_body(v_hbm, b_hbm, part_hbm, vbuf, ibuf, zbuf, acc):
  c = lax.axis_index("c")
  s = lax.axis_index("s")
  wid = s * NC + c

  zeros16 = jnp.zeros((16,), jnp.float32)
  def zrow(i, _):
    for j in range(D // 16):
      zbuf[i, pl.ds(j * 16, 16)] = zeros16
    return 0
  lax.fori_loop(0, PCH, zrow, 0)
  for t in range(NPCH):
    pltpu.sync_copy(zbuf, acc.at[pl.ds(s * RPS + t * PCH, PCH)])
  plsc.subcore_barrier()

  pltpu.sync_copy(b_hbm.at[wid], ibuf)

  r0 = wid * RPW
  def chunk(k, _):
    pltpu.sync_copy(v_hbm.at[pl.ds(r0 + k * C, C)], vbuf)
    pltpu.sync_copy(vbuf, acc.at[ibuf.at[k]], add=True)
    return 0
  lax.fori_loop(0, NCHUNK, chunk, 0)
  plsc.subcore_barrier()

  for t in range(NPCH):
    row = s * RPS + t * PCH
    pltpu.sync_copy(acc.at[pl.ds(row, PCH)], zbuf)
    pltpu.sync_copy(zbuf, part_hbm.at[c, pl.ds(row, PCH)])


_sc_scatter = pl.kernel(
    _sc_body,
    out_type=jax.ShapeDtypeStruct((NC, SP, D), jnp.float32),
    mesh=plsc.VectorSubcoreMesh(core_axis_name="c", subcore_axis_name="s"),
    scratch_types=[
        pltpu.VMEM((C, D), jnp.float32),
        pltpu.VMEM((NCHUNK, C), jnp.int32),
        pltpu.VMEM((PCH, D), jnp.float32),
        pltpu.VMEM_SHARED((SP, D), jnp.float32),
    ],
)


def _combine_body(u_ref, p_ref, o_ref):
  o_ref[...] = u_ref[...] + p_ref[0] + p_ref[1]


_combine = pl.pallas_call(
    _combine_body,
    grid=(10,),
    in_specs=[
        pl.BlockSpec((1000, D), lambda i: (i, 0)),
        pl.BlockSpec((NC, 1000, D), lambda i: (0, i, 0)),
    ],
    out_specs=pl.BlockSpec((1000, D), lambda i: (i, 0)),
    out_shape=jax.ShapeDtypeStruct((S, D), jnp.float32),
)


@jax.jit
def kernel(u, v, batch):
  b3 = batch.reshape(NW, NCHUNK, C)
  parts = _sc_scatter(v, b3)
  return _combine(u, parts)


# speedup vs baseline: 7.3434x; 1.6052x over previous
import jax
import jax.numpy as jnp
from jax import lax
from jax.experimental import pallas as pl
from jax.experimental.pallas import tpu as pltpu
from jax.experimental.pallas import tpu_sc as plsc

NC = 2
NS = 16
NW = NC * NS

S = 10000
SP = 10240
N = 320000
D = 128

C = 80
RPW = N // NW
NIT = RPW // C
RPS = SP // NS


def _sc_body(v_hbm, b_hbm, part_hbm, vbuf_a, vbuf_b, ibuf, acc,
             sem_in, sem_ib, sem_sc):
  c = lax.axis_index("c")
  s = lax.axis_index("s")
  wid = s * NC + c
  r0 = wid * RPW

  pltpu.async_copy(v_hbm.at[pl.ds(r0, C)], vbuf_a, sem_in)
  d_ibuf = pltpu.async_copy(b_hbm.at[wid], ibuf, sem_ib)

  zeros16 = jnp.zeros((16,), jnp.float32)
  def zrow(i, _):
    for j in range(D // 16):
      vbuf_b[i, pl.ds(j * 16, 16)] = zeros16
    return 0
  lax.fori_loop(0, C, zrow, 0)
  for t in range(RPS // C):
    pltpu.sync_copy(vbuf_b, acc.at[pl.ds(s * RPS + t * C, C)])
  d_ibuf.wait()
  plsc.subcore_barrier()

  def step(g, cur, nxt):
    @pl.when(g >= 1)
    def _():
      pltpu.make_async_copy(v_hbm.at[pl.ds(0, C)], nxt, sem_sc).wait()
    @pl.when(g + 1 < NIT)
    def _():
      pltpu.async_copy(v_hbm.at[pl.ds(r0 + (g + 1) * C, C)], nxt, sem_in)
    pltpu.make_async_copy(v_hbm.at[pl.ds(0, C)], cur, sem_in).wait()
    pltpu.async_copy(cur, acc.at[ibuf.at[g]], sem_sc, add=True)

  def chunk(g, _):
    @pl.when(g % 2 == 0)
    def _():
      step(g, vbuf_a, vbuf_b)
    @pl.when(g % 2 == 1)
    def _():
      step(g, vbuf_b, vbuf_a)
    return 0
  lax.fori_loop(0, NIT, chunk, 0)
  pltpu.make_async_copy(v_hbm.at[pl.ds(0, C)], vbuf_a, sem_sc).wait()
  plsc.subcore_barrier()

  for t in range(RPS // C):
    row = s * RPS + t * C
    pltpu.sync_copy(acc.at[pl.ds(row, C)], vbuf_a)
    pltpu.sync_copy(vbuf_a, part_hbm.at[c, pl.ds(row, C)])


_sc_scatter = pl.kernel(
    _sc_body,
    out_type=jax.ShapeDtypeStruct((NC, SP, D), jnp.float32),
    mesh=plsc.VectorSubcoreMesh(core_axis_name="c", subcore_axis_name="s"),
    scratch_types=[
        pltpu.VMEM((C, D), jnp.float32),
        pltpu.VMEM((C, D), jnp.float32),
        pltpu.VMEM((NIT, C), jnp.int32),
        pltpu.VMEM_SHARED((SP, D), jnp.float32),
        pltpu.SemaphoreType.DMA,
        pltpu.SemaphoreType.DMA,
        pltpu.SemaphoreType.DMA,
    ],
)


def _combine_body(u_ref, p_ref, o_ref):
  o_ref[...] = u_ref[...] + p_ref[0] + p_ref[1]


_combine = pl.pallas_call(
    _combine_body,
    grid=(10,),
    in_specs=[
        pl.BlockSpec((1000, D), lambda i: (i, 0)),
        pl.BlockSpec((NC, 1000, D), lambda i: (0, i, 0)),
    ],
    out_specs=pl.BlockSpec((1000, D), lambda i: (i, 0)),
    out_shape=jax.ShapeDtypeStruct((S, D), jnp.float32),
)


@jax.jit
def kernel(u, v, batch):
  b3 = batch.reshape(NW, NIT, C)
  parts = _sc_scatter(v, b3)
  return _combine(u, parts)


# speedup vs baseline: 8.2724x; 1.1265x over previous
import jax
import jax.numpy as jnp
from jax import lax
from jax.experimental import pallas as pl
from jax.experimental.pallas import tpu as pltpu
from jax.experimental.pallas import tpu_sc as plsc

NC = 2
NS = 16
NW = NC * NS

S = 10000
SP = 10240
N = 320000
D = 128

C = 80
RPW = N // NW
NIT = RPW // C
RPS = SP // NS
NB = 3


def _sc_body(v_hbm, b_hbm, part_hbm, vbuf_a, vbuf_b, vbuf_c, ibuf, acc,
             sem_in, sem_ib, sem_z, sem_sc):
  c = lax.axis_index("c")
  s = lax.axis_index("s")
  wid = s * NC + c
  r0 = wid * RPW
  bufs = [vbuf_a, vbuf_b, vbuf_c]

  for g in range(NB - 1):
    pltpu.async_copy(v_hbm.at[pl.ds(r0 + g * C, C)], bufs[g], sem_in)
  d_ibuf = pltpu.async_copy(b_hbm.at[wid], ibuf, sem_ib)

  zeros16 = jnp.zeros((16,), jnp.float32)
  def zrow(i, _):
    for j in range(D // 16):
      vbuf_c[i, pl.ds(j * 16, 16)] = zeros16
    return 0
  lax.fori_loop(0, C, zrow, 0)
  zdescs = [
      pltpu.async_copy(vbuf_c, acc.at[pl.ds(s * RPS + t * C, C)], sem_z)
      for t in range(RPS // C)
  ]
  for d in zdescs:
    d.wait()
  d_ibuf.wait()
  plsc.subcore_barrier()

  def step(g, cur, reuse):
    @pl.when(g >= 1)
    def _():
      pltpu.make_async_copy(v_hbm.at[pl.ds(0, C)], reuse, sem_sc).wait()
    @pl.when(g + NB - 1 < NIT)
    def _():
      pltpu.async_copy(v_hbm.at[pl.ds(r0 + (g + NB - 1) * C, C)], reuse,
                       sem_in)
    pltpu.make_async_copy(v_hbm.at[pl.ds(0, C)], cur, sem_in).wait()
    pltpu.async_copy(cur, acc.at[ibuf.at[g]], sem_sc, add=True)

  def chunk(g, _):
    for b in range(NB):
      @pl.when(g % NB == b)
      def _():
        step(g, bufs[b], bufs[(b + NB - 1) % NB])
    return 0
  lax.fori_loop(0, NIT, chunk, 0)
  pltpu.make_async_copy(v_hbm.at[pl.ds(0, C)], vbuf_a, sem_sc).wait()
  plsc.subcore_barrier()

  pltpu.sync_copy(acc.at[pl.ds(s * RPS, RPS)],
                  part_hbm.at[c, pl.ds(s * RPS, RPS)])


_sc_scatter = pl.kernel(
    _sc_body,
    out_type=jax.ShapeDtypeStruct((NC, SP, D), jnp.float32),
    mesh=plsc.VectorSubcoreMesh(core_axis_name="c", subcore_axis_name="s"),
    scratch_types=[
        pltpu.VMEM((C, D), jnp.float32),
        pltpu.VMEM((C, D), jnp.float32),
        pltpu.VMEM((C, D), jnp.float32),
        pltpu.VMEM((NIT, C), jnp.int32),
        pltpu.VMEM_SHARED((SP, D), jnp.float32),
        pltpu.SemaphoreType.DMA,
        pltpu.SemaphoreType.DMA,
        pltpu.SemaphoreType.DMA,
        pltpu.SemaphoreType.DMA,
    ],
)


def _combine_body(u_ref, p_ref, o_ref):
  o_ref[...] = u_ref[...] + p_ref[0] + p_ref[1]


_combine = pl.pallas_call(
    _combine_body,
    grid=(10,),
    in_specs=[
        pl.BlockSpec((1000, D), lambda i: (i, 0)),
        pl.BlockSpec((NC, 1000, D), lambda i: (0, i, 0)),
    ],
    out_specs=pl.BlockSpec((1000, D), lambda i: (i, 0)),
    out_shape=jax.ShapeDtypeStruct((S, D), jnp.float32),
)


@jax.jit
def kernel(u, v, batch):
  b3 = batch.reshape(NW, NIT, C)
  parts = _sc_scatter(v, b3)
  return _combine(u, parts)
